# baseline (speedup 1.0000x reference)
import functools

import jax
import jax.numpy as jnp
from jax import lax
from jax.experimental import pallas as pl
from jax.experimental.pallas import tpu as pltpu
from jax.experimental.pallas import tpu_sc as plsc

NC = 2
NS = 16
L = 16
NW = NC * NS

B = 16384
D = 32
BPW = B // NW
CH = 128
NCH = BPW // CH

_mesh = plsc.VectorSubcoreMesh(
    core_axis_name="c", subcore_axis_name="s", num_cores=NC, num_subcores=NS
)


@functools.partial(
    pl.kernel,
    out_type=jax.ShapeDtypeStruct((B, D), jnp.float32),
    mesh=_mesh,
    scratch_types=[
        pltpu.VMEM((2 * BPW,), jnp.int32),
        pltpu.VMEM((NCH, CH), jnp.int32),
        pltpu.VMEM((NCH, CH), jnp.int32),
        pltpu.VMEM((BPW, D), jnp.float32),
        pltpu.VMEM((BPW, D), jnp.float32),
        pltpu.SemaphoreType.DMA,
    ],
    compiler_params=pltpu.CompilerParams(use_tc_tiling_on_sc=False,
                                        needs_layout_passes=False),
)
def _gmf_sc(x_hbm, ut_hbm, it_hbm, out_hbm,
            xv, uidx, iidx, urows, irows, sem):
    wid = lax.axis_index("s") * NC + lax.axis_index("c")
    base = wid * BPW

    pltpu.sync_copy(x_hbm.at[wid], xv)

    lane = lax.iota(jnp.int32, 16)
    copies = []
    for c in range(NCH):
        for k in range(CH // L):
            flat = 2 * (c * CH + k * L) + 2 * lane
            uidx[c, pl.ds(k * L, L)] = plsc.load_gather(xv, [flat])
            iidx[c, pl.ds(k * L, L)] = plsc.load_gather(xv, [flat + 1])
        copies.append(
            pltpu.async_copy(ut_hbm.at[uidx.at[c]],
                             urows.at[pl.ds(c * CH, CH)], sem))
        copies.append(
            pltpu.async_copy(it_hbm.at[iidx.at[c]],
                             irows.at[pl.ds(c * CH, CH)], sem))
    for cp in copies:
        cp.wait()

    def body(r, _):
        for h in range(D // L):
            sl = pl.ds(h * L, L)
            urows[r, sl] = urows[r, sl] * irows[r, sl]
        return ()

    lax.fori_loop(0, BPW, body, (), unroll=4)

    pltpu.sync_copy(urows, out_hbm.at[pl.ds(base, BPW)])


def kernel(x, user_table, item_table):
    xw = x.astype(jnp.int32).reshape(NW, 2 * BPW)
    return _gmf_sc(xw, user_table, item_table)
